# baseline (speedup 1.0000x reference)
import functools

import jax
import jax.numpy as jnp
from jax import lax
from jax.experimental import pallas as pl
from jax.experimental.pallas import tpu as pltpu
from jax.experimental.pallas import tpu_sc as plsc

C, H, W = 192, 512, 512
N_SITES = 181
N_PAD16 = 192
C_BLK = 12

_MESH = plsc.VectorSubcoreMesh(core_axis_name="c", subcore_axis_name="s")


@functools.partial(
    pl.kernel,
    mesh=_MESH,
    out_type=jax.ShapeDtypeStruct((W,), jnp.float32),
    scratch_types=[
        pltpu.VMEM((N_PAD16,), jnp.int32),
        pltpu.VMEM((W,), jnp.float32),
    ],
    compiler_params=pltpu.CompilerParams(needs_layout_passes=False),
)
def _mask_sc(sites_hbm, mask_hbm, idx_v, mask_v):
    cid = lax.axis_index("c")
    sid = lax.axis_index("s")

    @pl.when((cid == 0) & (sid == 0))
    def _():
        pltpu.sync_copy(sites_hbm, idx_v)
        ones = jnp.ones((16,), jnp.float32)
        for i in range(W // 16):
            mask_v[pl.ds(i * 16, 16)] = ones
        zeros = jnp.zeros((16,), jnp.float32)
        for j in range(N_PAD16 // 16):
            idx = idx_v[pl.ds(j * 16, 16)]
            plsc.store_scatter(mask_v, [idx], zeros)
        pltpu.sync_copy(mask_v, mask_hbm)


def _mul_kernel(mask_ref, x_ref, o_ref):
    keep = mask_ref[...]
    o_ref[...] = x_ref[...] * keep[None, None, :]


def kernel(x, mask_sites):
    sites = mask_sites.astype(jnp.int32)
    sites = jnp.concatenate([sites, sites[: N_PAD16 - N_SITES]])
    mask = _mask_sc(sites)
    grid = (C // C_BLK,)
    return pl.pallas_call(
        _mul_kernel,
        grid=grid,
        in_specs=[
            pl.BlockSpec((W,), lambda i: (0,)),
            pl.BlockSpec((C_BLK, H, W), lambda i: (i, 0, 0)),
        ],
        out_specs=pl.BlockSpec((C_BLK, H, W), lambda i: (i, 0, 0)),
        out_shape=jax.ShapeDtypeStruct((C, H, W), jnp.float32),
    )(mask, x)


# speedup vs baseline: 1.1638x; 1.1638x over previous
import jax
import jax.numpy as jnp
from jax.experimental import pallas as pl
from jax.experimental.pallas import tpu as pltpu

C, H, W = 192, 512, 512
N_SITES = 181
C_BLK = 12


def _mask_mul_kernel(sites_ref, x_ref, o_ref, mask_scr):
    @pl.when(pl.program_id(0) == 0)
    def _():
        cols = jax.lax.broadcasted_iota(jnp.int32, (1, W), 1)

        def body(i, m):
            return jnp.where(cols == sites_ref[i], 0.0, m)

        mask_scr[...] = jax.lax.fori_loop(
            0, N_SITES, body, jnp.ones((1, W), jnp.float32)
        )

    o_ref[...] = x_ref[...] * mask_scr[...][None]


def kernel(x, mask_sites):
    grid = (C // C_BLK,)
    return pl.pallas_call(
        _mask_mul_kernel,
        grid=grid,
        in_specs=[
            pl.BlockSpec(memory_space=pltpu.SMEM),
            pl.BlockSpec((C_BLK, H, W), lambda i: (i, 0, 0)),
        ],
        out_specs=pl.BlockSpec((C_BLK, H, W), lambda i: (i, 0, 0)),
        out_shape=jax.ShapeDtypeStruct((C, H, W), jnp.float32),
        scratch_shapes=[pltpu.VMEM((1, W), jnp.float32)],
    )(mask_sites.astype(jnp.int32), x)
